# baseline (speedup 1.0000x reference)
import jax
import jax.numpy as jnp
from jax.experimental import pallas as pl

T_BLK = 1024
S_BLK = 2048


def _argmin_kernel(tar_ref, src_ref, tsq_ref, ssq_ref, out_ref):
    t = tar_ref[...]
    tsq = tsq_ref[...]
    n_src = src_ref.shape[1]

    def body(c, carry):
        bv, bi = carry
        s = src_ref[:, pl.ds(c * S_BLK, S_BLK)]
        mm = jax.lax.dot_general(
            t, s, (((1,), (0,)), ((), ())),
            preferred_element_type=jnp.float32)
        ssq = ssq_ref[:, pl.ds(c * S_BLK, S_BLK)]
        d2 = (tsq - 2.0 * mm) + ssq
        lane = jax.lax.broadcasted_iota(jnp.int32, (T_BLK, S_BLK), 1)
        ci = (lane + S_BLK * c).astype(jnp.float32)
        mask = d2 < bv
        bv = jnp.where(mask, d2, bv)
        bi = jnp.where(mask, ci, bi)
        return bv, bi

    bv0 = jnp.full((T_BLK, S_BLK), jnp.inf, jnp.float32)
    bi0 = jnp.zeros((T_BLK, S_BLK), jnp.float32)
    bv, bi = jax.lax.fori_loop(0, n_src // S_BLK, body, (bv0, bi0))

    vmin = jnp.min(bv, axis=1, keepdims=True)
    cand = jnp.where(bv == vmin, bi, jnp.float32(1e9))
    idx = jnp.min(cand, axis=1)
    out_ref[...] = idx.astype(jnp.int32).reshape(T_BLK, 1)


def _nn_indices_pallas(src_V, tar_V):
    n_src = src_V.shape[0]
    n_tar = tar_V.shape[0]
    tsq = jnp.sum(tar_V * tar_V, axis=1).reshape(n_tar, 1)
    ssq = jnp.sum(src_V * src_V, axis=1).reshape(1, n_src)
    tar_bf = tar_V.astype(jnp.bfloat16)
    src_bf = src_V.T.astype(jnp.bfloat16)
    idx = pl.pallas_call(
        _argmin_kernel,
        grid=(n_tar // T_BLK,),
        in_specs=[
            pl.BlockSpec((T_BLK, 3), lambda i: (i, 0)),
            pl.BlockSpec((3, n_src), lambda i: (0, 0)),
            pl.BlockSpec((T_BLK, 1), lambda i: (i, 0)),
            pl.BlockSpec((1, n_src), lambda i: (0, 0)),
        ],
        out_specs=pl.BlockSpec((T_BLK, 1), lambda i: (i, 0)),
        out_shape=jax.ShapeDtypeStruct((n_tar, 1), jnp.int32),
    )(tar_bf, src_bf, tsq, ssq)
    return idx[:, 0]


def kernel(src_V, tar_V):
    idx = _nn_indices_pallas(src_V, tar_V)
    g = jnp.take(src_V, idx, axis=0) - tar_V
    return 0.5 * jnp.sum(g * g)


# speedup vs baseline: 2.2363x; 2.2363x over previous
import jax
import jax.numpy as jnp
from jax.experimental import pallas as pl
from jax.experimental.pallas import tpu as pltpu

T_BLK = 1024
S_BLK = 2048


def _argmin_kernel(tar_ref, src_ref, tsq_ref, ssq_ref, out_ref,
                   bv_ref, bc_ref):
    t = tar_ref[...]
    tsq = tsq_ref[...]
    n_src = src_ref.shape[1]

    bv_ref[...] = jnp.full((T_BLK, S_BLK), jnp.inf, jnp.float32)
    bc_ref[...] = jnp.zeros((T_BLK, S_BLK), jnp.float32)

    def body(c, _):
        s = src_ref[:, pl.ds(c * S_BLK, S_BLK)]
        mm2 = jax.lax.dot_general(
            t, s, (((1,), (0,)), ((), ())),
            preferred_element_type=jnp.float32)
        ssq = ssq_ref[:, pl.ds(c * S_BLK, S_BLK)]
        d2 = (tsq + mm2) + ssq
        bv = bv_ref[...]
        mask = d2 < bv
        bv_ref[...] = jnp.where(mask, d2, bv)
        bc_ref[...] = jnp.where(mask, jnp.float32(c), bc_ref[...])
        return 0

    jax.lax.fori_loop(0, n_src // S_BLK, body, 0)

    bv = bv_ref[...]
    vmin = jnp.min(bv, axis=1, keepdims=True)
    lane = jax.lax.broadcasted_iota(jnp.int32, (T_BLK, S_BLK), 1)
    flat = bc_ref[...] * jnp.float32(S_BLK) + lane.astype(jnp.float32)
    cand = jnp.where(bv == vmin, flat, jnp.float32(1e9))
    idx = jnp.min(cand, axis=1)
    out_ref[...] = idx.astype(jnp.int32).reshape(T_BLK, 1)


def _nn_indices_pallas(src_V, tar_V):
    n_src = src_V.shape[0]
    n_tar = tar_V.shape[0]
    tsq = jnp.sum(tar_V * tar_V, axis=1).reshape(n_tar, 1)
    ssq = jnp.sum(src_V * src_V, axis=1).reshape(1, n_src)
    tar_bf = (-2.0 * tar_V).astype(jnp.bfloat16)
    src_bf = src_V.T.astype(jnp.bfloat16)
    idx = pl.pallas_call(
        _argmin_kernel,
        grid=(n_tar // T_BLK,),
        in_specs=[
            pl.BlockSpec((T_BLK, 3), lambda i: (i, 0)),
            pl.BlockSpec((3, n_src), lambda i: (0, 0)),
            pl.BlockSpec((T_BLK, 1), lambda i: (i, 0)),
            pl.BlockSpec((1, n_src), lambda i: (0, 0)),
        ],
        out_specs=pl.BlockSpec((T_BLK, 1), lambda i: (i, 0)),
        out_shape=jax.ShapeDtypeStruct((n_tar, 1), jnp.int32),
        scratch_shapes=[
            pltpu.VMEM((T_BLK, S_BLK), jnp.float32),
            pltpu.VMEM((T_BLK, S_BLK), jnp.float32),
        ],
    )(tar_bf, src_bf, tsq, ssq)
    return idx[:, 0]


def kernel(src_V, tar_V):
    idx = _nn_indices_pallas(src_V, tar_V)
    g = jnp.take(src_V, idx, axis=0) - tar_V
    return 0.5 * jnp.sum(g * g)
